# Initial kernel scaffold; baseline (speedup 1.0000x reference)
import jax
import jax.numpy as jnp
from jax.experimental import pallas as pl


def kernel(x, edge_index, graph_ids, mask_nodes, enc_params, con_params, proj_params, mask_token):
    raise NotImplementedError("write your pallas kernel here")


# speedup vs baseline: 3.0190x; 3.0190x over previous
import functools

import jax
import jax.numpy as jnp
from jax import lax
from jax.experimental import pallas as pl
from jax.experimental.pallas import tpu as pltpu
from jax.experimental.pallas import tpu_sc as plsc

_TEMP = 0.2
_NC = 2
_NS = 16


def _mask_apply(x, mask_nodes, mask_token):
    N, D = x.shape
    M = mask_nodes.shape[0]
    CPR = 200
    n_copy = -(-N // CPR)
    copy_per_tile = -(-n_copy // _NS)
    MCH = 128
    n_sc = -(-M // MCH)
    sc_per_tile = -(-n_sc // _NS)
    mesh = plsc.VectorSubcoreMesh(core_axis_name="c", subcore_axis_name="s")

    @functools.partial(
        pl.kernel,
        out_type=jax.ShapeDtypeStruct((N, D), jnp.float32),
        mesh=mesh,
        scratch_types=[
            pltpu.VMEM((CPR, D), jnp.float32),
            pltpu.VMEM((MCH, D), jnp.float32),
            pltpu.VMEM((MCH,), jnp.int32),
            pltpu.VMEM((1, D), jnp.float32),
        ],
    )
    def k(x_hbm, mi_hbm, tok_hbm, xm_hbm, buf_v, trows_v, idx_v, tok_v):
        cid = lax.axis_index("c")
        sid = lax.axis_index("s")
        on0 = cid == 0
        for t in range(copy_per_tile):
            q = sid + _NS * t

            @pl.when(jnp.logical_and(on0, q < n_copy))
            def _():
                off = q * CPR
                pltpu.sync_copy(x_hbm.at[pl.ds(off, CPR)], buf_v)
                pltpu.sync_copy(buf_v, xm_hbm.at[pl.ds(off, CPR)])

        plsc.subcore_barrier()

        @pl.when(on0)
        def _():
            pltpu.sync_copy(tok_hbm, tok_v)

            def fill(j, carry):
                for kk in range(D // 16):
                    trows_v[j, pl.ds(kk * 16, 16)] = tok_v[0, pl.ds(kk * 16, 16)]
                return carry

            lax.fori_loop(0, MCH, fill, 0)
            for t in range(sc_per_tile):
                g = sid + _NS * t

                @pl.when(g < n_sc)
                def _():
                    start = jnp.minimum(g * MCH, M - MCH)
                    pltpu.sync_copy(mi_hbm.at[pl.ds(start, MCH)], idx_v)
                    pltpu.sync_copy(trows_v, xm_hbm.at[idx_v])

    return k(x, mask_nodes, mask_token)


def _edge_segsum(h, src, dst, zrows):
    N, D = h.shape
    E = src.shape[0]
    NW = _NC * _NS
    EP = E // NW
    CH = 80
    n_it = EP // CH
    RPT = (N // _NS) // 8 * 8
    TAIL = N - _NS * RPT
    mesh = plsc.VectorSubcoreMesh(core_axis_name="c", subcore_axis_name="s")

    @functools.partial(
        pl.kernel,
        out_type=jax.ShapeDtypeStruct((_NC * N, D), jnp.float32),
        mesh=mesh,
        scratch_types=[
            pltpu.VMEM_SHARED((N, D), jnp.float32),
            pltpu.VMEM((CH,), jnp.int32),
            pltpu.VMEM((CH,), jnp.int32),
            pltpu.VMEM((CH, D), jnp.float32),
            pltpu.SemaphoreType.DMA,
        ],
    )
    def k(h_hbm, src_hbm, dst_hbm, z_hbm, out_hbm, acc_sh, si_v, di_v, rows_v, sem):
        cid = lax.axis_index("c")
        sid = lax.axis_index("s")
        wid = sid * _NC + cid
        pltpu.sync_copy(z_hbm, acc_sh.at[pl.ds(sid * RPT, RPT)])
        if TAIL:
            @pl.when(sid == 0)
            def _():
                pltpu.sync_copy(z_hbm.at[pl.ds(0, TAIL)],
                                acc_sh.at[pl.ds(_NS * RPT, TAIL)])
        plsc.subcore_barrier()
        e0 = wid * EP

        def body(i, carry):
            e = e0 + i * CH
            pltpu.sync_copy(src_hbm.at[pl.ds(e, CH)], si_v)
            pltpu.sync_copy(dst_hbm.at[pl.ds(e, CH)], di_v)
            pltpu.async_copy(h_hbm.at[si_v], rows_v, sem).wait()
            pltpu.sync_copy(rows_v, acc_sh.at[di_v], add=True)
            return carry

        lax.fori_loop(0, n_it, body, 0)
        plsc.subcore_barrier()
        pltpu.sync_copy(
            acc_sh.at[pl.ds(sid * RPT, RPT)],
            out_hbm.at[pl.ds(cid * N + sid * RPT, RPT)],
        )
        if TAIL:
            @pl.when(sid == 0)
            def _():
                pltpu.sync_copy(
                    acc_sh.at[pl.ds(_NS * RPT, TAIL)],
                    out_hbm.at[pl.ds(cid * N + _NS * RPT, TAIL)],
                )

    return k(h, src, dst, zrows)


def _dense_layer(h, agg2, p, gid2):
    N, D = h.shape
    Bg = 16
    Hh = p["W1"].shape[0]

    def body(h_ref, a_ref, w1_ref, mg_ref, mb_ref, w2_ref, g_ref, b_ref, gid_ref,
             ho_ref, pool_ref):
        h_ = h_ref[...]
        h2 = h_ + a_ref[0:N] + a_ref[N:2 * N]
        y = lax.dot_general(h2, w1_ref[...], (((1,), (1,)), ((), ())),
                            preferred_element_type=jnp.float32)
        mu = jnp.mean(y, axis=0, keepdims=True)
        var = jnp.mean((y - mu) ** 2, axis=0, keepdims=True)
        y = (y - mu) * lax.rsqrt(var + 1e-5) * mg_ref[...] + mb_ref[...]
        y = jnp.maximum(y, 0.0)
        z = lax.dot_general(y, w2_ref[...], (((1,), (1,)), ((), ())),
                            preferred_element_type=jnp.float32)
        mu2 = jnp.mean(z, axis=0, keepdims=True)
        var2 = jnp.mean((z - mu2) ** 2, axis=0, keepdims=True)
        z = (z - mu2) * lax.rsqrt(var2 + 1e-5) * g_ref[...] + b_ref[...]
        hn = jnp.maximum(z, 0.0)
        ho_ref[...] = hn
        oh = (gid_ref[...] == lax.broadcasted_iota(jnp.int32, (1, Bg), 1)).astype(
            jnp.float32)
        pool_ref[...] = lax.dot_general(oh, hn, (((0,), (0,)), ((), ())),
                                        preferred_element_type=jnp.float32)

    return pl.pallas_call(
        body,
        out_shape=(
            jax.ShapeDtypeStruct((N, p["W2"].shape[0]), jnp.float32),
            jax.ShapeDtypeStruct((Bg, p["W2"].shape[0]), jnp.float32),
        ),
    )(h, agg2, p["W1"], p["mbn_g"].reshape(1, Hh), p["mbn_b"].reshape(1, Hh),
      p["W2"], p["bn_g"].reshape(1, -1), p["bn_b"].reshape(1, -1), gid2)


def _head(ch, gh, pp):
    def body(ch_ref, gh_ref, w1_ref, b1_ref, w2_ref, b2_ref, out_ref):
        def proj(z):
            z1 = lax.dot_general(z, w1_ref[...], (((1,), (1,)), ((), ())),
                                 preferred_element_type=jnp.float32) + b1_ref[...]
            z1 = jnp.maximum(z1, 0.0)
            return lax.dot_general(z1, w2_ref[...], (((1,), (1,)), ((), ())),
                                   preferred_element_type=jnp.float32) + b2_ref[...]

        c_h = proj(ch_ref[...])
        c_m = proj(gh_ref[...])
        na = jnp.sqrt(jnp.sum(c_h * c_h, axis=1, keepdims=True))
        nb = jnp.sqrt(jnp.sum(c_m * c_m, axis=1, keepdims=True))
        outer = lax.dot_general(na, nb, (((1,), (1,)), ((), ())),
                                preferred_element_type=jnp.float32)
        sim = jnp.exp(
            lax.dot_general(c_h, c_m, (((1,), (1,)), ((), ())),
                            preferred_element_type=jnp.float32) / outer / _TEMP)
        Bg = sim.shape[0]
        eye = (lax.broadcasted_iota(jnp.int32, (Bg, Bg), 0)
               == lax.broadcasted_iota(jnp.int32, (Bg, Bg), 1)).astype(jnp.float32)
        pos = jnp.sum(sim * eye, axis=1, keepdims=True)
        tot = jnp.sum(sim, axis=1, keepdims=True)
        lvec = jnp.log(pos / (tot - pos))
        out_ref[...] = -jnp.mean(lvec) * jnp.ones((1, 1), jnp.float32)

    return pl.pallas_call(
        body,
        out_shape=jax.ShapeDtypeStruct((1, 1), jnp.float32),
    )(ch, gh, pp["W1"], pp["b1"].reshape(1, -1), pp["W2"], pp["b2"].reshape(1, -1))


def kernel(x, edge_index, graph_ids, mask_nodes, enc_params, con_params,
           proj_params, mask_token):
    N, D = x.shape
    src = edge_index[0].astype(jnp.int32)
    dst = edge_index[1].astype(jnp.int32)
    mask_nodes = mask_nodes.astype(jnp.int32)
    gid2 = graph_ids.astype(jnp.int32).reshape(N, 1)
    zrows = jnp.zeros(((N // _NS) // 8 * 8, D), jnp.float32)

    xm = _mask_apply(x, mask_nodes, mask_token)

    def encoder(h0, params):
        h = h0
        pools = []
        for p in params:
            agg2 = _edge_segsum(h, src, dst, zrows)
            h, pool = _dense_layer(h, agg2, p, gid2)
            pools.append(pool)
        return h, jnp.concatenate(pools, axis=1)

    _, gh = encoder(xm, enc_params)
    _, ch = encoder(x, con_params)
    out = _head(ch, gh, proj_params)
    return out[0, 0]
